# baseline (speedup 1.0000x reference)
import functools

import jax
import jax.numpy as jnp
from jax import lax
from jax.experimental import pallas as pl
from jax.experimental.pallas import tpu as pltpu
from jax.experimental.pallas import tpu_sc as plsc

B = 4
M = 8192
D = 768
NC = 2
NS = 16
NW = NC * NS
POS_PER_W = M // NW
CH = 32
HH = CH // 2
CHUNKS = POS_PER_W // CH
VPR = D // 16
NH = CHUNKS * B * 2
NB = 6
LA = 4


def _pos_add_body(x_hbm, pos_hbm, out_hbm,
                  xh0, xh1, xh2, xh3, xh4, xh5, pv0, pv1,
                  si0, si1, si2, si3, si4, si5,
                  so0, so1, so2, so3, so4, so5,
                  sp0, sp1, spb0, spb1):
    wid = lax.axis_index("s") * NC + lax.axis_index("c")
    row0 = wid * POS_PER_W
    xh = [xh0, xh1, xh2, xh3, xh4, xh5]
    pv = [pv0, pv1]
    sin = [si0, si1, si2, si3, si4, si5]
    sout = [so0, so1, so2, so3, so4, so5]
    sp = [sp0, sp1]
    spb = [spb0, spb1]

    def x_row(h):
        g, q = divmod(h, 2)
        c, b = divmod(g, B)
        return b * M + row0 + c * CH + q * HH

    def issue_pos(c):
        rp = row0 + c * CH
        pa = pltpu.async_copy(pos_hbm.at[pl.ds(rp, HH)],
                              pv[c % 2].at[pl.ds(0, HH)], sp[c % 2])
        pb = pltpu.async_copy(pos_hbm.at[pl.ds(rp + HH, HH)],
                              pv[c % 2].at[pl.ds(HH, HH)], spb[c % 2])
        return pa, pb

    def issue_in(h):
        return pltpu.async_copy(x_hbm.at[pl.ds(x_row(h), HH)],
                                xh[h % NB], sin[h % NB])

    in_h = [None] * NH
    out_h = [None] * NH
    pos_h = [None] * CHUNKS

    pos_h[0] = issue_pos(0)
    for h in range(LA):
        in_h[h] = issue_in(h)

    for h in range(NH):
        s = h % NB
        g, q = divmod(h, 2)
        c = g // B
        if h % (2 * B) == 0 and c + 1 < CHUNKS:
            pos_h[c + 1] = issue_pos(c + 1)
        if h + LA < NH:
            if h + LA >= NB:
                out_h[h + LA - NB].wait()
            in_h[h + LA] = issue_in(h + LA)
        if h % (2 * B) == 0:
            pos_h[c][0].wait()
        elif h % (2 * B) == 1:
            pos_h[c][1].wait()
        in_h[h].wait()

        pvs = pv[c % 2]
        xhs = xh[s]
        poff = q * HH

        @plsc.parallel_loop(0, HH, unroll=1)
        def _row_body(i):
            @plsc.parallel_loop(0, VPR, unroll=4)
            def _vec_body(v):
                j = v * 16
                plsc.addupdate(xhs.at[i, pl.ds(j, 16)],
                               pvs[poff + i, pl.ds(j, 16)])

        out_h[h] = pltpu.async_copy(xhs, out_hbm.at[pl.ds(x_row(h), HH)],
                                    sout[s])

    for h in range(NH - NB, NH):
        out_h[h].wait()


_pos_add = functools.partial(
    pl.kernel,
    out_type=jax.ShapeDtypeStruct((B * M, D), jnp.float32),
    mesh=plsc.VectorSubcoreMesh(core_axis_name="c", subcore_axis_name="s"),
    scratch_types=[
        pltpu.VMEM((HH, D), jnp.float32),
        pltpu.VMEM((HH, D), jnp.float32),
        pltpu.VMEM((HH, D), jnp.float32),
        pltpu.VMEM((HH, D), jnp.float32),
        pltpu.VMEM((HH, D), jnp.float32),
        pltpu.VMEM((HH, D), jnp.float32),
        pltpu.VMEM((CH, D), jnp.float32),
        pltpu.VMEM((CH, D), jnp.float32),
    ] + [pltpu.SemaphoreType.DMA] * 16,
)(_pos_add_body)


@jax.jit
def kernel(x, pos_table):
    out = _pos_add(x.reshape(B * M, D), pos_table)
    return out.reshape(x.shape)


# speedup vs baseline: 1.0130x; 1.0130x over previous
import functools

import jax
import jax.numpy as jnp
from jax import lax
from jax.experimental import pallas as pl
from jax.experimental.pallas import tpu as pltpu
from jax.experimental.pallas import tpu_sc as plsc

B = 4
M = 8192
D = 768
NC = 2
NS = 16
NW = NC * NS
POS_PER_W = M // NW
CH = 32
HH = CH // 2
CHUNKS = POS_PER_W // CH
VPR = D // 16
NIT = CHUNKS * B
NBUF = 3


def _pos_add_body(x_hbm, pos_hbm, out_hbm,
                  xv0, xv1, xv2, pv0, pv1,
                  sa0, sa1, sa2, sb0, sb1, sb2,
                  soa0, soa1, soa2, sob0, sob1, sob2, sp0, sp1, spb0, spb1):
    wid = lax.axis_index("s") * NC + lax.axis_index("c")
    row0 = wid * POS_PER_W
    xv = [xv0, xv1, xv2]
    pv = [pv0, pv1]
    sina = [sa0, sa1, sa2]
    sinb = [sb0, sb1, sb2]
    souta = [soa0, soa1, soa2]
    soutb = [sob0, sob1, sob2]
    sp = [sp0, sp1]
    spb = [spb0, spb1]

    def x_row(g):
        c, b = divmod(g, B)
        return b * M + row0 + c * CH

    def issue_in(g):
        s = g % NBUF
        r = x_row(g)
        ha = pltpu.async_copy(x_hbm.at[pl.ds(r, HH)],
                              xv[s].at[pl.ds(0, HH)], sina[s])
        hb = pltpu.async_copy(x_hbm.at[pl.ds(r + HH, HH)],
                              xv[s].at[pl.ds(HH, HH)], sinb[s])
        return ha, hb

    in_h = [None] * NIT
    out_h = [None] * NIT
    pos_h = [None] * CHUNKS

    def issue_pos(c):
        rp = row0 + c * CH
        pa = pltpu.async_copy(pos_hbm.at[pl.ds(rp, HH)],
                              pv[c % 2].at[pl.ds(0, HH)], sp[c % 2])
        pb = pltpu.async_copy(pos_hbm.at[pl.ds(rp + HH, HH)],
                              pv[c % 2].at[pl.ds(HH, HH)], spb[c % 2])
        return pa, pb

    pos_h[0] = issue_pos(0)
    in_h[0] = issue_in(0)
    in_h[1] = issue_in(1)

    for g in range(NIT):
        s = g % NBUF
        c = g // B
        if g % B == 0 and c + 1 < CHUNKS:
            pos_h[c + 1] = issue_pos(c + 1)
        if g % B == 0:
            pos_h[c][0].wait()

        pvs = pv[c % 2]
        xvs = xv[s]
        r = x_row(g)

        in_h[g][0].wait()

        @plsc.parallel_loop(0, HH, unroll=1)
        def _row_body_a(i):
            @plsc.parallel_loop(0, VPR, unroll=4)
            def _vec_body(v):
                j = v * 16
                plsc.addupdate(xvs.at[i, pl.ds(j, 16)], pvs[i, pl.ds(j, 16)])

        oa = pltpu.async_copy(xvs.at[pl.ds(0, HH)],
                              out_hbm.at[pl.ds(r, HH)], souta[s])

        if g + 2 < NIT:
            s2 = (g + 2) % NBUF
            r2 = x_row(g + 2)
            if g >= 1:
                out_h[g - 1][0].wait()
            ia = pltpu.async_copy(x_hbm.at[pl.ds(r2, HH)],
                                  xv[s2].at[pl.ds(0, HH)], sina[s2])

        if g % B == 0:
            pos_h[c][1].wait()
        in_h[g][1].wait()

        @plsc.parallel_loop(HH, CH, unroll=1)
        def _row_body_b(i):
            @plsc.parallel_loop(0, VPR, unroll=4)
            def _vec_body(v):
                j = v * 16
                plsc.addupdate(xvs.at[i, pl.ds(j, 16)], pvs[i, pl.ds(j, 16)])

        if g + 2 < NIT:
            if g >= 1:
                out_h[g - 1][1].wait()
            ib = pltpu.async_copy(x_hbm.at[pl.ds(r2 + HH, HH)],
                                  xv[s2].at[pl.ds(HH, HH)], sinb[s2])
            in_h[g + 2] = (ia, ib)

        ob = pltpu.async_copy(xvs.at[pl.ds(HH, HH)],
                              out_hbm.at[pl.ds(r + HH, HH)], soutb[s])
        out_h[g] = (oa, ob)

    for g in (NIT - 2, NIT - 1):
        out_h[g][0].wait()
        out_h[g][1].wait()


_pos_add = functools.partial(
    pl.kernel,
    out_type=jax.ShapeDtypeStruct((B * M, D), jnp.float32),
    mesh=plsc.VectorSubcoreMesh(core_axis_name="c", subcore_axis_name="s"),
    scratch_types=[
        pltpu.VMEM((CH, D), jnp.float32),
        pltpu.VMEM((CH, D), jnp.float32),
        pltpu.VMEM((CH, D), jnp.float32),
        pltpu.VMEM((CH, D), jnp.float32),
        pltpu.VMEM((CH, D), jnp.float32),
        pltpu.SemaphoreType.DMA,
        pltpu.SemaphoreType.DMA,
        pltpu.SemaphoreType.DMA,
        pltpu.SemaphoreType.DMA,
        pltpu.SemaphoreType.DMA,
        pltpu.SemaphoreType.DMA,
        pltpu.SemaphoreType.DMA,
        pltpu.SemaphoreType.DMA,
        pltpu.SemaphoreType.DMA,
        pltpu.SemaphoreType.DMA,
        pltpu.SemaphoreType.DMA,
        pltpu.SemaphoreType.DMA,
        pltpu.SemaphoreType.DMA,
        pltpu.SemaphoreType.DMA,
        pltpu.SemaphoreType.DMA,
        pltpu.SemaphoreType.DMA,
    ],
)(_pos_add_body)


@jax.jit
def kernel(x, pos_table):
    out = _pos_add(x.reshape(B * M, D), pos_table)
    return out.reshape(x.shape)
